# baseline (speedup 1.0000x reference)
import math

import jax
import jax.numpy as jnp
from jax.experimental import pallas as pl
from jax.experimental.pallas import tpu as pltpu

N_EMBD = 64
MEMORY_SIZE = 16384
BQ = 512


def _attn_kernel(x_ref, mem_ref, wq_ref, bq_ref, wk_ref, bk_ref,
                 wv_ref, bv_ref, wo_ref, bo_ref, out_ref,
                 k_scratch, kt_scratch, v_scratch, vt_scratch, knorm_ref):
    @pl.when(pl.program_id(0) == 0)
    def _project_kv():
        m = mem_ref[...]
        nm = m.shape[0]
        k = m @ wk_ref[...].T + bk_ref[...]
        knorm_ref[0] = jnp.sqrt(jnp.max(jnp.sum(k * k, axis=1)))
        kb = k.astype(jnp.bfloat16)
        k_scratch[...] = kb
        kt_scratch[...] = kb.T
        v = m @ wv_ref[...].T + bv_ref[...]
        v_aug = jnp.concatenate(
            [v, jnp.ones((nm, 1), jnp.float32), jnp.zeros((nm, 63), jnp.float32)],
            axis=1)
        vb = v_aug.astype(jnp.bfloat16)
        v_scratch[...] = vb
        vt_scratch[...] = vb.T

    scale = (1.0 / math.sqrt(N_EMBD)) * 1.4426950408889634
    q = (x_ref[...] @ wq_ref[...].T + bq_ref[...]) * scale
    s_bound = jnp.sqrt(jnp.sum(q * q, axis=1, keepdims=True)) * knorm_ref[0]
    qb = q.astype(jnp.bfloat16)
    hr = qb.shape[0] // 2
    s1 = jax.lax.dot_general(qb[:hr], k_scratch[...],
                             (((1,), (1,)), ((), ())),
                             preferred_element_type=jnp.float32)
    s2 = jax.lax.dot_general(qb[hr:], kt_scratch[...],
                             (((1,), (0,)), ((), ())),
                             preferred_element_type=jnp.float32)
    p1 = jnp.exp2((s1 - s_bound[:hr]).astype(jnp.bfloat16))
    p2 = jnp.exp2((s2 - s_bound[hr:]).astype(jnp.bfloat16))
    ctx1 = jax.lax.dot_general(p1, v_scratch[...],
                               (((1,), (0,)), ((), ())),
                               preferred_element_type=jnp.float32)
    ctx2 = jax.lax.dot_general(p2, vt_scratch[...],
                               (((1,), (1,)), ((), ())),
                               preferred_element_type=jnp.float32)
    ctx = jnp.concatenate([ctx1, ctx2], axis=0)
    o = ctx[:, :N_EMBD] @ wo_ref[...].T
    l = ctx[:, N_EMBD:N_EMBD + 1]
    out_ref[...] = o / l + bo_ref[...]


def kernel(x, memory, Wq, bq, Wk, bk, Wv, bv, Wo, bo):
    b, s, d = x.shape
    nq = b * s
    xf = x.reshape(nq, d)
    bq2 = bq.reshape(1, d)
    bk2 = bk.reshape(1, d)
    bv2 = bv.reshape(1, d)
    bo2 = bo.reshape(1, d)

    grid = (nq // BQ,)
    row_spec = pl.BlockSpec((BQ, d), lambda i: (i, 0))
    full = lambda shape: pl.BlockSpec(shape, lambda i: (0,) * len(shape))

    out = pl.pallas_call(
        _attn_kernel,
        grid=grid,
        in_specs=[
            row_spec,
            full((MEMORY_SIZE, d)),
            full((d, d)), full((1, d)),
            full((d, d)), full((1, d)),
            full((d, d)), full((1, d)),
            full((d, d)), full((1, d)),
        ],
        out_specs=row_spec,
        out_shape=jax.ShapeDtypeStruct((nq, d), x.dtype),
        scratch_shapes=[
            pltpu.VMEM((MEMORY_SIZE, N_EMBD), jnp.bfloat16),
            pltpu.VMEM((N_EMBD, MEMORY_SIZE), jnp.bfloat16),
            pltpu.VMEM((MEMORY_SIZE, 128), jnp.bfloat16),
            pltpu.VMEM((128, MEMORY_SIZE), jnp.bfloat16),
            pltpu.SMEM((1,), jnp.float32),
        ],
        compiler_params=pltpu.CompilerParams(
            dimension_semantics=("arbitrary",),
        ),
    )(xf, memory, Wq, bq2, Wk, bk2, Wv, bv2, Wo, bo2)
    return out.reshape(b, s, d)


# speedup vs baseline: 1.0251x; 1.0251x over previous
import math

import jax
import jax.numpy as jnp
from jax.experimental import pallas as pl
from jax.experimental.pallas import tpu as pltpu

N_EMBD = 64
MEMORY_SIZE = 16384
BQ = 1024


def _attn_kernel(x_ref, mem_ref, wq_ref, bq_ref, wk_ref, bk_ref,
                 wv_ref, bv_ref, wo_ref, bo_ref, out_ref,
                 k_scratch, v_scratch, knorm_ref):
    @pl.when(pl.program_id(0) == 0)
    def _project_kv():
        m = mem_ref[...]
        nm = m.shape[0]
        k = m @ wk_ref[...].T + bk_ref[...]
        knorm_ref[0] = jnp.sqrt(jnp.max(jnp.sum(k * k, axis=1)))
        k_scratch[...] = k.astype(jnp.bfloat16)
        v = m @ wv_ref[...].T + bv_ref[...]
        v_aug = jnp.concatenate(
            [v, jnp.ones((nm, 1), jnp.float32), jnp.zeros((nm, 63), jnp.float32)],
            axis=1)
        v_scratch[...] = v_aug.astype(jnp.bfloat16)

    scale = (1.0 / math.sqrt(N_EMBD)) * 1.4426950408889634
    q = (x_ref[...] @ wq_ref[...].T + bq_ref[...]) * scale
    s_bound = jnp.sqrt(jnp.sum(q * q, axis=1, keepdims=True)) * knorm_ref[0]
    s = jax.lax.dot_general(q.astype(jnp.bfloat16), k_scratch[...],
                            (((1,), (1,)), ((), ())),
                            preferred_element_type=jnp.float32)
    p = jnp.exp2(s - s_bound).astype(jnp.bfloat16)
    ctx = jax.lax.dot_general(p, v_scratch[...],
                              (((1,), (0,)), ((), ())),
                              preferred_element_type=jnp.float32)
    o = ctx[:, :N_EMBD] @ wo_ref[...].T
    l = ctx[:, N_EMBD:N_EMBD + 1]
    out_ref[...] = o / l + bo_ref[...]


def kernel(x, memory, Wq, bq, Wk, bk, Wv, bv, Wo, bo):
    b, s, d = x.shape
    nq = b * s
    xf = x.reshape(nq, d)
    bq2 = bq.reshape(1, d)
    bk2 = bk.reshape(1, d)
    bv2 = bv.reshape(1, d)
    bo2 = bo.reshape(1, d)

    grid = (nq // BQ,)
    row_spec = pl.BlockSpec((BQ, d), lambda i: (i, 0))
    full = lambda shape: pl.BlockSpec(shape, lambda i: (0,) * len(shape))

    out = pl.pallas_call(
        _attn_kernel,
        grid=grid,
        in_specs=[
            row_spec,
            full((MEMORY_SIZE, d)),
            full((d, d)), full((1, d)),
            full((d, d)), full((1, d)),
            full((d, d)), full((1, d)),
            full((d, d)), full((1, d)),
        ],
        out_specs=row_spec,
        out_shape=jax.ShapeDtypeStruct((nq, d), x.dtype),
        scratch_shapes=[
            pltpu.VMEM((MEMORY_SIZE, N_EMBD), jnp.bfloat16),
            pltpu.VMEM((MEMORY_SIZE, 128), jnp.bfloat16),
            pltpu.SMEM((1,), jnp.float32),
        ],
        compiler_params=pltpu.CompilerParams(
            dimension_semantics=("arbitrary",),
        ),
    )(xf, memory, Wq, bq2, Wk, bk2, Wv, bv2, Wo, bo2)
    return out.reshape(b, s, d)


# speedup vs baseline: 1.0363x; 1.0109x over previous
import math

import jax
import jax.numpy as jnp
from jax.experimental import pallas as pl
from jax.experimental.pallas import tpu as pltpu

N_EMBD = 64
MEMORY_SIZE = 16384
BQ = 1024


def _attn_kernel(x_ref, mem_ref, wq_ref, bq_ref, wk_ref, bk_ref,
                 wv_ref, bv_ref, wo_ref, bo_ref, out_ref,
                 k_scratch, v_scratch, knorm_ref):
    @pl.when(pl.program_id(0) == 0)
    def _project_kv():
        m = mem_ref[...]
        nm = m.shape[0]
        k = m @ wk_ref[...].T + bk_ref[...]
        knorm_ref[0] = jnp.sqrt(jnp.max(jnp.sum(k * k, axis=1)))
        k_scratch[...] = k.astype(jnp.bfloat16)
        v = m @ wv_ref[...].T + bv_ref[...]
        v_aug = jnp.concatenate(
            [v, jnp.ones((nm, 1), jnp.float32), jnp.zeros((nm, 63), jnp.float32)],
            axis=1)
        v_scratch[...] = v_aug.astype(jnp.bfloat16)

    scale = (1.0 / math.sqrt(N_EMBD)) * 1.4426950408889634
    q = (x_ref[...] @ wq_ref[...].T + bq_ref[...]) * scale
    s_bound = jnp.sqrt(jnp.sum(q * q, axis=1, keepdims=True)) * knorm_ref[0]
    s = jax.lax.dot_general(q.astype(jnp.bfloat16), k_scratch[...],
                            (((1,), (1,)), ((), ())),
                            preferred_element_type=jnp.float32)
    p = jnp.exp2((s - s_bound).astype(jnp.bfloat16))
    ctx = jax.lax.dot_general(p, v_scratch[...],
                              (((1,), (0,)), ((), ())),
                              preferred_element_type=jnp.float32)
    o = ctx[:, :N_EMBD] @ wo_ref[...].T
    l = ctx[:, N_EMBD:N_EMBD + 1]
    out_ref[...] = o / l + bo_ref[...]


def kernel(x, memory, Wq, bq, Wk, bk, Wv, bv, Wo, bo):
    b, s, d = x.shape
    nq = b * s
    xf = x.reshape(nq, d)
    bq2 = bq.reshape(1, d)
    bk2 = bk.reshape(1, d)
    bv2 = bv.reshape(1, d)
    bo2 = bo.reshape(1, d)

    grid = (nq // BQ,)
    row_spec = pl.BlockSpec((BQ, d), lambda i: (i, 0))
    full = lambda shape: pl.BlockSpec(shape, lambda i: (0,) * len(shape))

    out = pl.pallas_call(
        _attn_kernel,
        grid=grid,
        in_specs=[
            row_spec,
            full((MEMORY_SIZE, d)),
            full((d, d)), full((1, d)),
            full((d, d)), full((1, d)),
            full((d, d)), full((1, d)),
            full((d, d)), full((1, d)),
        ],
        out_specs=row_spec,
        out_shape=jax.ShapeDtypeStruct((nq, d), x.dtype),
        scratch_shapes=[
            pltpu.VMEM((MEMORY_SIZE, N_EMBD), jnp.bfloat16),
            pltpu.VMEM((MEMORY_SIZE, 128), jnp.bfloat16),
            pltpu.SMEM((1,), jnp.float32),
        ],
        compiler_params=pltpu.CompilerParams(
            dimension_semantics=("arbitrary",),
        ),
    )(xf, memory, Wq, bq2, Wk, bk2, Wv, bv2, Wo, bo2)
    return out.reshape(b, s, d)
